# Initial kernel scaffold; baseline (speedup 1.0000x reference)
import jax
import jax.numpy as jnp
from jax.experimental import pallas as pl


def kernel(xyz, new_xyz, features):
    raise NotImplementedError("write your pallas kernel here")


# speedup vs baseline: 22.6495x; 22.6495x over previous
import functools

import jax
import jax.numpy as jnp
from jax import lax
from jax.experimental import pallas as pl
from jax.experimental.pallas import tpu as pltpu
from jax.experimental.pallas import tpu_sc as plsc

K = 16
B = 8
N = 4096
M = 1024
C = 128

QBLK = 512
INT_MAX = 0x7FFFFFFF
IDX_BITS = 10
IDX_MASK = (1 << IDX_BITS) - 1


def _topk_weights_kernel(x0, x1, x2, y0, y1, y2, idx_out, w_out):
    b = pl.program_id(0)
    xq0 = x0[0]
    xq1 = x1[0]
    xq2 = x2[0]
    yr0 = y0[0]
    yr1 = y1[0]
    yr2 = y2[0]
    def bf(v):
        return v.astype(jnp.bfloat16).astype(jnp.float32)

    x2 = xq0 * xq0 + xq1 * xq1 + xq2 * xq2
    y2 = yr0 * yr0 + yr1 * yr1 + yr2 * yr2
    xy = (bf(xq0) * bf(yr0) + bf(xq1) * bf(yr1)
          + bf(xq2) * bf(yr2))
    d2 = jnp.maximum((x2 + y2) - 2.0 * xy, 0.0)

    bits = lax.bitcast_convert_type(d2, jnp.int32)
    bits = (bits + (1 << (IDX_BITS - 1))) & ~IDX_MASK
    col = lax.broadcasted_iota(jnp.int32, (QBLK, M), 1)
    key = lax.bitcast_convert_type((bits | col) + (1 << 23), jnp.float32)

    kprev = jnp.full((QBLK, 1), -1.0, jnp.float32)
    picks = []
    for _ in range(K):
        cand = jnp.where(key > kprev, key, 3.4e38)
        kmin = jnp.min(cand, axis=1, keepdims=True)
        picks.append(kmin)
        kprev = kmin
    kcat = lax.bitcast_convert_type(
        jnp.concatenate(picks, axis=1), jnp.int32) - (1 << 23)

    sel_idx = kcat & IDX_MASK
    sel_d2 = lax.bitcast_convert_type(kcat & ~IDX_MASK, jnp.float32)
    dist = jnp.sqrt(sel_d2)
    dist = jnp.maximum(dist, 1e-10)
    w = 1.0 / dist
    w = w / jnp.sum(w, axis=1, keepdims=True)

    idx_out[0] = sel_idx + b * M
    w_out[0] = w


def _topk_weights(xyz, new_xyz):
    nb = xyz.shape[0]
    grid = (nb, N // QBLK)
    x_cols = [xyz[:, :, c].reshape(nb, N, 1) for c in range(3)]
    y_rows = [new_xyz[:, :, c].reshape(nb, 1, M) for c in range(3)]
    x_spec = pl.BlockSpec((1, QBLK, 1), lambda b, n: (b, n, 0))
    y_spec = pl.BlockSpec((1, 1, M), lambda b, n: (b, 0, 0))
    o_spec = pl.BlockSpec((1, QBLK, K), lambda b, n: (b, n, 0))
    idx, w = pl.pallas_call(
        _topk_weights_kernel,
        grid=grid,
        in_specs=[x_spec, x_spec, x_spec, y_spec, y_spec, y_spec],
        out_specs=[o_spec, o_spec],
        out_shape=[
            jax.ShapeDtypeStruct((nb, N, K), jnp.int32),
            jax.ShapeDtypeStruct((nb, N, K), jnp.float32),
        ],
    )(*x_cols, *y_rows)
    return idx.reshape(nb * N, K), w.reshape(nb * N, K)


NW = 32
QW = (B * N) // NW
G = 8


def _bcast_lane(v, k):
    return lax.gather(
        v, jnp.full((16, 1), k, jnp.int32),
        lax.GatherDimensionNumbers(offset_dims=(), collapsed_slice_dims=(0,),
                                   start_index_map=(0,)),
        slice_sizes=(1,), mode=lax.GatherScatterMode.PROMISE_IN_BOUNDS)


def _make_gather_kernel(qw):
  def _gather_kernel(feat_hbm, idx_hbm, w_hbm, out_hbm,
                     idx_v, w_v, rows0, rows1, out0, out1,
                     gsem0, gsem1, osem0, osem1):
    nc = lax.axis_size("c")
    wid = lax.axis_index("s") * nc + lax.axis_index("c")
    base = wid * qw
    ngroups = qw // G
    rows = (rows0, rows1)
    outs = (out0, out1)
    gsems = (gsem0, gsem1)
    osems = (osem0, osem1)

    pltpu.sync_copy(idx_hbm.at[pl.ds(base * K, qw * K)], idx_v)
    pltpu.sync_copy(w_hbm.at[pl.ds(base * K, qw * K)], w_v)

    def fire(g, par):
        for q in range(G):
            pltpu.async_copy(feat_hbm.at[idx_v.at[pl.ds((g * G + q) * K, K)]],
                             rows[par].at[pl.ds(q * K, K)], gsems[par])

    def drain_gathers(g, par):
        for q in range(G):
            pltpu.make_async_copy(feat_hbm.at[idx_v.at[pl.ds((g * G + q) * K, K)]],
                                  rows[par].at[pl.ds(q * K, K)],
                                  gsems[par]).wait()

    def compute(g, par):
        for q in range(G):
            wv = w_v[pl.ds((g * G + q) * K, K)]
            accs = [jnp.zeros((16,), jnp.float32) for _ in range(C // 16)]
            for k in range(K):
                wk = _bcast_lane(wv, k)
                for j in range(C // 16):
                    accs[j] = accs[j] + wk * rows[par][q * K + k,
                                                       pl.ds(j * 16, 16)]
            for j in range(C // 16):
                outs[par][q, pl.ds(j * 16, 16)] = accs[j]

    fire(0, 0)

    def pair(i, _):
        for par in range(2):
            g = 2 * i + par
            nxt = 1 - par

            @pl.when(g + 1 < ngroups)
            def _():
                fire(g + 1, nxt)

            drain_gathers(g, par)

            @pl.when(g >= 2)
            def _():
                pltpu.make_async_copy(
                    outs[par], out_hbm.at[pl.ds(base + (g - 2) * G, G)],
                    osems[par]).wait()

            compute(g, par)
            pltpu.async_copy(outs[par],
                             out_hbm.at[pl.ds(base + g * G, G)], osems[par])
        return ()

    lax.fori_loop(0, ngroups // 2, pair, (), unroll=False)
    for par in range(2):
        g = ngroups - 2 + par
        pltpu.make_async_copy(outs[par],
                              out_hbm.at[pl.ds(base + g * G, G)],
                              osems[par]).wait()

  return _gather_kernel


def _weighted_gather(feat, idx, w):
    nq = idx.shape[0]
    qw = nq // NW
    mesh = plsc.VectorSubcoreMesh(core_axis_name="c", subcore_axis_name="s")
    kern = pl.kernel(
        _make_gather_kernel(qw),
        mesh=mesh,
        out_type=jax.ShapeDtypeStruct((nq, C), jnp.float32),
        scratch_types=[
            pltpu.VMEM((qw * K,), jnp.int32),
            pltpu.VMEM((qw * K,), jnp.float32),
            pltpu.VMEM((G * K, C), jnp.float32),
            pltpu.VMEM((G * K, C), jnp.float32),
            pltpu.VMEM((G, C), jnp.float32),
            pltpu.VMEM((G, C), jnp.float32),
            pltpu.SemaphoreType.DMA,
            pltpu.SemaphoreType.DMA,
            pltpu.SemaphoreType.DMA,
            pltpu.SemaphoreType.DMA,
        ],
    )
    return kern(feat, idx.reshape(-1), w.reshape(-1))


def kernel(xyz, new_xyz, features):
    h = B // 4
    outs = []
    for s in range(4):
        sl = slice(s * h, (s + 1) * h)
        idx, w = _topk_weights(xyz[sl], new_xyz[sl])
        feat = features[sl].transpose(0, 2, 1).reshape(h * M, C)
        outs.append(_weighted_gather(feat, idx, w))
    out = jnp.concatenate([o.reshape(h, N, C) for o in outs], axis=0)
    return out.transpose(0, 2, 1)


# speedup vs baseline: 23.2086x; 1.0247x over previous
import functools

import jax
import jax.numpy as jnp
from jax import lax
from jax.experimental import pallas as pl
from jax.experimental.pallas import tpu as pltpu
from jax.experimental.pallas import tpu_sc as plsc

K = 16
B = 8
N = 4096
M = 1024
C = 128

QBLK = 512
INT_MAX = 0x7FFFFFFF
IDX_BITS = 10
IDX_MASK = (1 << IDX_BITS) - 1


def _topk_weights_kernel(x0, x1, x2, y0, y1, y2, idx_out, w_out):
    b = pl.program_id(0)
    xq0 = x0[0]
    xq1 = x1[0]
    xq2 = x2[0]
    yr0 = y0[0]
    yr1 = y1[0]
    yr2 = y2[0]
    def bf(v):
        return v.astype(jnp.bfloat16).astype(jnp.float32)

    x2 = xq0 * xq0 + xq1 * xq1 + xq2 * xq2
    y2 = yr0 * yr0 + yr1 * yr1 + yr2 * yr2
    xy = (bf(xq0) * bf(yr0) + bf(xq1) * bf(yr1)
          + bf(xq2) * bf(yr2))
    d2 = jnp.maximum((x2 + y2) - 2.0 * xy, 0.0)

    bits = lax.bitcast_convert_type(d2, jnp.int32)
    bits = (bits + (1 << (IDX_BITS - 1))) & ~IDX_MASK
    col = lax.broadcasted_iota(jnp.int32, (QBLK, M), 1)
    key = lax.bitcast_convert_type((bits | col) + (1 << 23), jnp.float32)

    kprev = jnp.full((QBLK, 1), -1.0, jnp.float32)
    picks = []
    for _ in range(K):
        cand = jnp.where(key > kprev, key, 3.4e38)
        kmin = jnp.min(cand, axis=1, keepdims=True)
        picks.append(kmin)
        kprev = kmin
    kcat = lax.bitcast_convert_type(
        jnp.concatenate(picks, axis=1), jnp.int32) - (1 << 23)

    sel_idx = kcat & IDX_MASK
    sel_d2 = lax.bitcast_convert_type(kcat & ~IDX_MASK, jnp.float32)
    dist = jnp.sqrt(sel_d2)
    dist = jnp.maximum(dist, 1e-10)
    w = 1.0 / dist
    w = w / jnp.sum(w, axis=1, keepdims=True)

    idx_out[0] = sel_idx + b * M
    w_out[0] = w


def _topk_weights(xyz, new_xyz):
    nb = xyz.shape[0]
    grid = (nb, N // QBLK)
    x_cols = [xyz[:, :, c].reshape(nb, N, 1) for c in range(3)]
    y_rows = [new_xyz[:, :, c].reshape(nb, 1, M) for c in range(3)]
    x_spec = pl.BlockSpec((1, QBLK, 1), lambda b, n: (b, n, 0))
    y_spec = pl.BlockSpec((1, 1, M), lambda b, n: (b, 0, 0))
    o_spec = pl.BlockSpec((1, QBLK, K), lambda b, n: (b, n, 0))
    idx, w = pl.pallas_call(
        _topk_weights_kernel,
        grid=grid,
        in_specs=[x_spec, x_spec, x_spec, y_spec, y_spec, y_spec],
        out_specs=[o_spec, o_spec],
        out_shape=[
            jax.ShapeDtypeStruct((nb, N, K), jnp.int32),
            jax.ShapeDtypeStruct((nb, N, K), jnp.float32),
        ],
    )(*x_cols, *y_rows)
    return idx.reshape(nb * N, K), w.reshape(nb * N, K)


NW = 32
QW = (B * N) // NW
G = 8


def _bcast_lane(v, k):
    return lax.gather(
        v, jnp.full((16, 1), k, jnp.int32),
        lax.GatherDimensionNumbers(offset_dims=(), collapsed_slice_dims=(0,),
                                   start_index_map=(0,)),
        slice_sizes=(1,), mode=lax.GatherScatterMode.PROMISE_IN_BOUNDS)


def _make_gather_kernel(qw):
  def _gather_kernel(feat_hbm, idx_hbm, w_hbm, out_hbm,
                     feat_sh, idx_v, w_v, rows0, rows1, out0, out1,
                     gsem0, gsem1, osem0, osem1):
    nc = lax.axis_size("c")
    sid = lax.axis_index("s")
    wid = sid * nc + lax.axis_index("c")
    base = wid * qw
    ngroups = qw // G
    rows = (rows0, rows1)
    outs = (out0, out1)
    gsems = (gsem0, gsem1)
    osems = (osem0, osem1)

    @pl.when(sid == 0)
    def _():
        pltpu.sync_copy(feat_hbm, feat_sh)
    plsc.subcore_barrier()

    pltpu.sync_copy(idx_hbm.at[pl.ds(base * K, qw * K)], idx_v)
    pltpu.sync_copy(w_hbm.at[pl.ds(base * K, qw * K)], w_v)

    def fire(g, par):
        for q in range(G):
            pltpu.async_copy(feat_sh.at[idx_v.at[pl.ds((g * G + q) * K, K)]],
                             rows[par].at[pl.ds(q * K, K)], gsems[par])

    def drain_gathers(g, par):
        for q in range(G):
            pltpu.make_async_copy(feat_sh.at[idx_v.at[pl.ds((g * G + q) * K, K)]],
                                  rows[par].at[pl.ds(q * K, K)],
                                  gsems[par]).wait()

    def compute(g, par):
        for q in range(G):
            wv = w_v[pl.ds((g * G + q) * K, K)]
            accs = [jnp.zeros((16,), jnp.float32) for _ in range(C // 16)]
            for k in range(K):
                wk = _bcast_lane(wv, k)
                for j in range(C // 16):
                    accs[j] = accs[j] + wk * rows[par][q * K + k,
                                                       pl.ds(j * 16, 16)]
            for j in range(C // 16):
                outs[par][q, pl.ds(j * 16, 16)] = accs[j]

    fire(0, 0)

    def pair(i, _):
        for par in range(2):
            g = 2 * i + par
            nxt = 1 - par

            @pl.when(g + 1 < ngroups)
            def _():
                fire(g + 1, nxt)

            drain_gathers(g, par)

            @pl.when(g >= 2)
            def _():
                pltpu.make_async_copy(
                    outs[par], out_hbm.at[pl.ds(base + (g - 2) * G, G)],
                    osems[par]).wait()

            compute(g, par)
            pltpu.async_copy(outs[par],
                             out_hbm.at[pl.ds(base + g * G, G)], osems[par])
        return ()

    lax.fori_loop(0, ngroups // 2, pair, (), unroll=False)
    for par in range(2):
        g = ngroups - 2 + par
        pltpu.make_async_copy(outs[par],
                              out_hbm.at[pl.ds(base + g * G, G)],
                              osems[par]).wait()

  return _gather_kernel


def _weighted_gather(feat, idx, w):
    nq = idx.shape[0]
    qw = nq // NW
    mesh = plsc.VectorSubcoreMesh(core_axis_name="c", subcore_axis_name="s")
    kern = pl.kernel(
        _make_gather_kernel(qw),
        mesh=mesh,
        out_type=jax.ShapeDtypeStruct((nq, C), jnp.float32),
        scratch_types=[
            pltpu.VMEM_SHARED((2 * M, C), jnp.float32),
            pltpu.VMEM((qw * K,), jnp.int32),
            pltpu.VMEM((qw * K,), jnp.float32),
            pltpu.VMEM((G * K, C), jnp.float32),
            pltpu.VMEM((G * K, C), jnp.float32),
            pltpu.VMEM((G, C), jnp.float32),
            pltpu.VMEM((G, C), jnp.float32),
            pltpu.SemaphoreType.DMA,
            pltpu.SemaphoreType.DMA,
            pltpu.SemaphoreType.DMA,
            pltpu.SemaphoreType.DMA,
        ],
    )
    return kern(feat, idx.reshape(-1), w.reshape(-1))


def kernel(xyz, new_xyz, features):
    h = B // 4
    outs = []
    for s in range(4):
        sl = slice(s * h, (s + 1) * h)
        idx, w = _topk_weights(xyz[sl], new_xyz[sl])
        feat = features[sl].transpose(0, 2, 1).reshape(h * M, C)
        outs.append(_weighted_gather(feat, idx, w))
    out = jnp.concatenate([o.reshape(h, N, C) for o in outs], axis=0)
    return out.transpose(0, 2, 1)


# speedup vs baseline: 23.3171x; 1.0047x over previous
import functools

import jax
import jax.numpy as jnp
from jax import lax
from jax.experimental import pallas as pl
from jax.experimental.pallas import tpu as pltpu
from jax.experimental.pallas import tpu_sc as plsc

K = 16
B = 8
N = 4096
M = 1024
C = 128

QBLK = 512
INT_MAX = 0x7FFFFFFF
IDX_BITS = 10
IDX_MASK = (1 << IDX_BITS) - 1


def _topk_weights_kernel(x0, x1, x2, y0, y1, y2, idx_out, w_out):
    b = pl.program_id(0)
    xq0 = x0[0]
    xq1 = x1[0]
    xq2 = x2[0]
    yr0 = y0[0]
    yr1 = y1[0]
    yr2 = y2[0]
    def bf(v):
        return v.astype(jnp.bfloat16).astype(jnp.float32)

    x2 = xq0 * xq0 + xq1 * xq1 + xq2 * xq2
    y2 = yr0 * yr0 + yr1 * yr1 + yr2 * yr2
    xy = (bf(xq0) * bf(yr0) + bf(xq1) * bf(yr1)
          + bf(xq2) * bf(yr2))
    d2 = jnp.maximum((x2 + y2) - 2.0 * xy, 0.0)

    bits = lax.bitcast_convert_type(d2, jnp.int32)
    bits = (bits + (1 << (IDX_BITS - 1))) & ~IDX_MASK
    col = lax.broadcasted_iota(jnp.int32, (QBLK, M), 1)
    key = lax.bitcast_convert_type((bits | col) + (1 << 23), jnp.float32)

    kprev = jnp.full((QBLK, 1), -1.0, jnp.float32)
    picks = []
    for _ in range(K):
        cand = jnp.where(key > kprev, key, 3.4e38)
        kmin = jnp.min(cand, axis=1, keepdims=True)
        picks.append(kmin)
        kprev = kmin
    kcat = lax.bitcast_convert_type(
        jnp.concatenate(picks, axis=1), jnp.int32) - (1 << 23)

    sel_idx = kcat & IDX_MASK
    sel_d2 = lax.bitcast_convert_type(kcat & ~IDX_MASK, jnp.float32)
    dist = jnp.sqrt(sel_d2)
    dist = jnp.maximum(dist, 1e-10)
    w = 1.0 / dist
    w = w / jnp.sum(w, axis=1, keepdims=True)

    idx_out[0] = sel_idx + b * M
    w_out[0] = w


def _topk_weights(xyz, new_xyz):
    nb = xyz.shape[0]
    grid = (nb, N // QBLK)
    x_cols = [xyz[:, :, c].reshape(nb, N, 1) for c in range(3)]
    y_rows = [new_xyz[:, :, c].reshape(nb, 1, M) for c in range(3)]
    x_spec = pl.BlockSpec((1, QBLK, 1), lambda b, n: (b, n, 0))
    y_spec = pl.BlockSpec((1, 1, M), lambda b, n: (b, 0, 0))
    o_spec = pl.BlockSpec((1, QBLK, K), lambda b, n: (b, n, 0))
    idx, w = pl.pallas_call(
        _topk_weights_kernel,
        grid=grid,
        in_specs=[x_spec, x_spec, x_spec, y_spec, y_spec, y_spec],
        out_specs=[o_spec, o_spec],
        out_shape=[
            jax.ShapeDtypeStruct((nb, N, K), jnp.int32),
            jax.ShapeDtypeStruct((nb, N, K), jnp.float32),
        ],
    )(*x_cols, *y_rows)
    return idx.reshape(nb * N, K), w.reshape(nb * N, K)


NW = 32
QW = (B * N) // NW
G = 8


def _bcast_lane(v, k):
    return lax.gather(
        v, jnp.full((16, 1), k, jnp.int32),
        lax.GatherDimensionNumbers(offset_dims=(), collapsed_slice_dims=(0,),
                                   start_index_map=(0,)),
        slice_sizes=(1,), mode=lax.GatherScatterMode.PROMISE_IN_BOUNDS)


def _make_gather_kernel(qw):
  def _gather_kernel(feat_hbm, idx_hbm, w_hbm, out_hbm,
                     feat_sh, idx_v, w_v, rows0, rows1, out0, out1,
                     gsem0, gsem1, osem0, osem1):
    nc = lax.axis_size("c")
    sid = lax.axis_index("s")
    wid = sid * nc + lax.axis_index("c")
    base = wid * qw
    ngroups = qw // G
    rows = (rows0, rows1)
    outs = (out0, out1)
    gsems = (gsem0, gsem1)
    osems = (osem0, osem1)

    @pl.when(sid == 0)
    def _():
        pltpu.sync_copy(feat_hbm, feat_sh)
    plsc.subcore_barrier()

    pltpu.sync_copy(idx_hbm.at[pl.ds(base * K, qw * K)], idx_v)
    pltpu.sync_copy(w_hbm.at[pl.ds(base * K, qw * K)], w_v)

    def fire(g, par):
        pltpu.async_copy(feat_sh.at[idx_v.at[pl.ds(g * G * K, G * K)]],
                         rows[par], gsems[par])

    def drain_gathers(g, par):
        pltpu.make_async_copy(feat_sh.at[idx_v.at[pl.ds(g * G * K, G * K)]],
                              rows[par], gsems[par]).wait()

    def compute(g, par):
        for q in range(G):
            wv = w_v[pl.ds((g * G + q) * K, K)]
            accs = [jnp.zeros((16,), jnp.float32) for _ in range(C // 16)]
            for k in range(K):
                wk = _bcast_lane(wv, k)
                for j in range(C // 16):
                    accs[j] = accs[j] + wk * rows[par][q * K + k,
                                                       pl.ds(j * 16, 16)]
            for j in range(C // 16):
                outs[par][q, pl.ds(j * 16, 16)] = accs[j]

    fire(0, 0)

    def pair(i, _):
        for par in range(2):
            g = 2 * i + par
            nxt = 1 - par

            @pl.when(g + 1 < ngroups)
            def _():
                fire(g + 1, nxt)

            drain_gathers(g, par)

            @pl.when(g >= 2)
            def _():
                pltpu.make_async_copy(
                    outs[par], out_hbm.at[pl.ds(base + (g - 2) * G, G)],
                    osems[par]).wait()

            compute(g, par)
            pltpu.async_copy(outs[par],
                             out_hbm.at[pl.ds(base + g * G, G)], osems[par])
        return ()

    lax.fori_loop(0, ngroups // 2, pair, (), unroll=False)
    for par in range(2):
        g = ngroups - 2 + par
        pltpu.make_async_copy(outs[par],
                              out_hbm.at[pl.ds(base + g * G, G)],
                              osems[par]).wait()

  return _gather_kernel


def _weighted_gather(feat, idx, w):
    nq = idx.shape[0]
    qw = nq // NW
    mesh = plsc.VectorSubcoreMesh(core_axis_name="c", subcore_axis_name="s")
    kern = pl.kernel(
        _make_gather_kernel(qw),
        mesh=mesh,
        out_type=jax.ShapeDtypeStruct((nq, C), jnp.float32),
        scratch_types=[
            pltpu.VMEM_SHARED((2 * M, C), jnp.float32),
            pltpu.VMEM((qw * K,), jnp.int32),
            pltpu.VMEM((qw * K,), jnp.float32),
            pltpu.VMEM((G * K, C), jnp.float32),
            pltpu.VMEM((G * K, C), jnp.float32),
            pltpu.VMEM((G, C), jnp.float32),
            pltpu.VMEM((G, C), jnp.float32),
            pltpu.SemaphoreType.DMA,
            pltpu.SemaphoreType.DMA,
            pltpu.SemaphoreType.DMA,
            pltpu.SemaphoreType.DMA,
        ],
    )
    return kern(feat, idx.reshape(-1), w.reshape(-1))


def kernel(xyz, new_xyz, features):
    h = B // 4
    outs = []
    for s in range(4):
        sl = slice(s * h, (s + 1) * h)
        idx, w = _topk_weights(xyz[sl], new_xyz[sl])
        feat = features[sl].transpose(0, 2, 1).reshape(h * M, C)
        outs.append(_weighted_gather(feat, idx, w))
    out = jnp.concatenate([o.reshape(h, N, C) for o in outs], axis=0)
    return out.transpose(0, 2, 1)
